# baseline (speedup 1.0000x reference)
import functools

import jax
import jax.numpy as jnp
from jax import lax
from jax.experimental import pallas as pl
from jax.experimental.pallas import tpu as pltpu
from jax.experimental.pallas import tpu_sc as plsc

EMBED_DIM = 64
IDX_MINOR = 128
CHUNK = 4
NBUF = 2
NC = 2
NS = 16
NW = NC * NS


def _build_lookup(n_idx, vocab):
    assert n_idx % (NW * IDX_MINOR) == 0
    rows_total = n_idx // IDX_MINOR
    rows_per_w = rows_total // NW
    assert rows_per_w % (CHUNK * NBUF) == 0
    n_chunks = rows_per_w // CHUNK
    ci = CHUNK * IDX_MINOR

    mesh = plsc.VectorSubcoreMesh(core_axis_name="c", subcore_axis_name="s")

    @functools.partial(
        pl.kernel,
        mesh=mesh,
        out_type=jax.ShapeDtypeStruct((n_idx, EMBED_DIM), jnp.float32),
        scratch_types=[
            pltpu.VMEM((rows_per_w, IDX_MINOR), jnp.int32),
            pltpu.VMEM((NBUF, ci, EMBED_DIM), jnp.float32),
            pltpu.SemaphoreType.DMA((NBUF,)),
            pltpu.SemaphoreType.DMA((NBUF,)),
        ],
        compiler_params=pltpu.CompilerParams(use_tc_tiling_on_sc=False),
    )
    def lookup(idx_hbm, tab_hbm, out_hbm, idx_all, rows_v, gsem, wsem):
        wid = lax.axis_index("s") * NC + lax.axis_index("c")
        row0 = wid * rows_per_w
        out0 = row0 * IDX_MINOR

        pltpu.sync_copy(idx_hbm.at[pl.ds(row0, rows_per_w)], idx_all)

        def fire_gathers(c, b):
            for i in range(CHUNK):
                pltpu.async_copy(
                    tab_hbm.at[idx_all.at[c * CHUNK + i]],
                    rows_v.at[b, pl.ds(i * IDX_MINOR, IDX_MINOR)],
                    gsem.at[b],
                )

        def wait_gathers(b):
            pltpu.make_async_copy(
                tab_hbm.at[pl.ds(0, ci)], rows_v.at[b], gsem.at[b]
            ).wait()

        def fire_write(c, b):
            pltpu.async_copy(
                rows_v.at[b], out_hbm.at[pl.ds(out0 + c * ci, ci)], wsem.at[b]
            )

        def wait_write(b):
            pltpu.make_async_copy(
                rows_v.at[b], out_hbm.at[pl.ds(0, ci)], wsem.at[b]
            ).wait()

        fire_gathers(0, 0)
        wait_gathers(0)
        fire_write(0, 0)
        fire_gathers(1, 1)

        def outer(t, _):
            c0 = NBUF * t + NBUF
            for b in range(NBUF):
                c = c0 + b
                wait_gathers(1 - b)
                fire_write(c - 1, 1 - b)
                wait_write(b)
                fire_gathers(c, b)
            return _

        lax.fori_loop(0, n_chunks // NBUF - 1, outer, None)

        last = n_chunks - 1
        wait_gathers(last % NBUF)
        fire_write(last, last % NBUF)
        wait_write((last - 1) % NBUF)
        wait_write(last % NBUF)

    return lookup


_LOOKUP = None


def kernel(x, table):
    global _LOOKUP
    n_idx = x.size
    if _LOOKUP is None:
        _LOOKUP = _build_lookup(n_idx, table.shape[0])
    idx2d = x.reshape(-1).astype(jnp.int32).reshape(n_idx // IDX_MINOR, IDX_MINOR)
    out = _LOOKUP(idx2d, table)
    return out.reshape(x.shape + (EMBED_DIM,))


# speedup vs baseline: 1.0012x; 1.0012x over previous
import functools

import jax
import jax.numpy as jnp
from jax import lax
from jax.experimental import pallas as pl
from jax.experimental.pallas import tpu as pltpu
from jax.experimental.pallas import tpu_sc as plsc

EMBED_DIM = 64
IDX_MINOR = 128
CHUNK = 2
NB = 6
K = 4
NC = 2
NS = 16
NW = NC * NS


def _build_lookup(n_idx, vocab):
    assert n_idx % (NW * IDX_MINOR) == 0
    rows_total = n_idx // IDX_MINOR
    rows_per_w = rows_total // NW
    assert rows_per_w % CHUNK == 0
    n_chunks = rows_per_w // CHUNK
    assert n_chunks > NB > K
    ci = CHUNK * IDX_MINOR

    mesh = plsc.VectorSubcoreMesh(core_axis_name="c", subcore_axis_name="s")

    @functools.partial(
        pl.kernel,
        mesh=mesh,
        out_type=jax.ShapeDtypeStruct((n_idx, EMBED_DIM), jnp.float32),
        scratch_types=[
            pltpu.VMEM((rows_per_w, IDX_MINOR), jnp.int32),
            pltpu.VMEM((NB, ci, EMBED_DIM), jnp.float32),
            pltpu.SemaphoreType.DMA((NB,)),
            pltpu.SemaphoreType.DMA((NB,)),
        ],
        compiler_params=pltpu.CompilerParams(use_tc_tiling_on_sc=False),
    )
    def lookup(idx_hbm, tab_hbm, out_hbm, idx_all, rows_v, gsem, wsem):
        wid = lax.axis_index("s") * NC + lax.axis_index("c")
        row0 = wid * rows_per_w
        out0 = row0 * IDX_MINOR

        pltpu.sync_copy(idx_hbm.at[pl.ds(row0, rows_per_w)], idx_all)

        def fire_gathers(c, b):
            for i in range(CHUNK):
                pltpu.async_copy(
                    tab_hbm.at[idx_all.at[c * CHUNK + i]],
                    rows_v.at[b, pl.ds(i * IDX_MINOR, IDX_MINOR)],
                    gsem.at[b],
                )

        def wait_gathers(b):
            pltpu.make_async_copy(
                tab_hbm.at[pl.ds(0, ci)], rows_v.at[b], gsem.at[b]
            ).wait()

        def fire_write(c, b):
            pltpu.async_copy(
                rows_v.at[b], out_hbm.at[pl.ds(out0 + c * ci, ci)], wsem.at[b]
            )

        def wait_write(b):
            pltpu.make_async_copy(
                rows_v.at[b], out_hbm.at[pl.ds(0, ci)], wsem.at[b]
            ).wait()

        for c in range(K):
            fire_gathers(c, c)
        for c in range(K, NB):
            fire_gathers(c, c)
            wait_gathers(c - K)
            fire_write(c - K, c - K)

        def body(c, _):
            b = lax.rem(c, NB)
            br = lax.rem(c - K, NB)
            wait_write(b)
            fire_gathers(c, b)
            wait_gathers(br)
            fire_write(c - K, br)
            return _

        lax.fori_loop(NB, n_chunks, body, None)

        for c in range(n_chunks - K, n_chunks):
            wait_gathers(c % NB)
            fire_write(c, c % NB)
        for c in range(n_chunks - NB, n_chunks):
            wait_write(c % NB)

    return lookup


_LOOKUP = None


def kernel(x, table):
    global _LOOKUP
    n_idx = x.size
    if _LOOKUP is None:
        _LOOKUP = _build_lookup(n_idx, table.shape[0])
    idx2d = x.reshape(-1).astype(jnp.int32).reshape(n_idx // IDX_MINOR, IDX_MINOR)
    out = _LOOKUP(idx2d, table)
    return out.reshape(x.shape + (EMBED_DIM,))
